# baseline (speedup 1.0000x reference)
import jax
import jax.numpy as jnp
from jax import lax
from jax.experimental import pallas as pl
from jax.experimental.pallas import tpu as pltpu
from jax.experimental.pallas import tpu_sc as plsc

EMBED_DIM = 64
BATCH = 16384

NC = 2
NS = 16
LANES = 16
NW = NC * NS
B_PER_W = BATCH // NW
CHUNK = 128
NCHUNK = B_PER_W // CHUNK


def _body(uidx_hbm, pidx_hbm, nidx_hbm, utab_hbm, itab_hbm,
          pos_hbm, neg_hbm,
          uidx_v, pidx_v, nidx_v, urows, prows, nrows, pos_v, neg_v, sem):
    wid = lax.axis_index("s") * NC + lax.axis_index("c")
    base = wid * B_PER_W

    pltpu.sync_copy(uidx_hbm.at[wid], uidx_v)
    pltpu.sync_copy(pidx_hbm.at[wid], pidx_v)
    pltpu.sync_copy(nidx_hbm.at[wid], nidx_v)

    copies = []
    for c in range(NCHUNK):
        copies.append(pltpu.async_copy(utab_hbm.at[uidx_v.at[c]], urows.at[c], sem))
        copies.append(pltpu.async_copy(itab_hbm.at[pidx_v.at[c]], prows.at[c], sem))
        copies.append(pltpu.async_copy(itab_hbm.at[nidx_v.at[c]], nrows.at[c], sem))
    for cp in copies:
        cp.wait()

    lane = lax.broadcasted_iota(jnp.int32, (LANES,), 0)
    zero = jnp.zeros((LANES,), jnp.float32)

    for c in range(NCHUNK):
        cvec = jnp.full((LANES,), c, jnp.int32)
        for g in range(CHUNK // LANES):
            row = g * LANES + lane

            def dstep(d, acc):
                pacc, nacc = acc
                col = (lane + d) & (EMBED_DIM - 1)
                lu = plsc.load_gather(urows, [cvec, row, col])
                lp = plsc.load_gather(prows, [cvec, row, col])
                ln = plsc.load_gather(nrows, [cvec, row, col])
                return pacc + lu * lp, nacc + lu * ln

            pacc, nacc = lax.fori_loop(0, EMBED_DIM, dstep, (zero, zero))
            off = c * CHUNK + g * LANES
            pos_v[pl.ds(off, LANES)] = pacc
            neg_v[pl.ds(off, LANES)] = nacc

    pltpu.sync_copy(pos_v, pos_hbm.at[pl.ds(base, B_PER_W)])
    pltpu.sync_copy(neg_v, neg_hbm.at[pl.ds(base, B_PER_W)])


@jax.jit
def kernel(user_inputs, pos_item_inputs, neg_item_inputs, user_table, item_table):
    mesh = plsc.VectorSubcoreMesh(core_axis_name="c", subcore_axis_name="s")
    uidx = user_inputs.astype(jnp.int32).reshape(NW, NCHUNK, CHUNK)
    pidx = pos_item_inputs.astype(jnp.int32).reshape(NW, NCHUNK, CHUNK)
    nidx = neg_item_inputs.astype(jnp.int32).reshape(NW, NCHUNK, CHUNK)
    run = pl.kernel(
        _body,
        out_type=(jax.ShapeDtypeStruct((BATCH,), jnp.float32),
                  jax.ShapeDtypeStruct((BATCH,), jnp.float32)),
        mesh=mesh,
        compiler_params=pltpu.CompilerParams(
            use_tc_tiling_on_sc=False, needs_layout_passes=False),
        scratch_types=[
            pltpu.VMEM((NCHUNK, CHUNK), jnp.int32),
            pltpu.VMEM((NCHUNK, CHUNK), jnp.int32),
            pltpu.VMEM((NCHUNK, CHUNK), jnp.int32),
            pltpu.VMEM((NCHUNK, CHUNK, EMBED_DIM), jnp.float32),
            pltpu.VMEM((NCHUNK, CHUNK, EMBED_DIM), jnp.float32),
            pltpu.VMEM((NCHUNK, CHUNK, EMBED_DIM), jnp.float32),
            pltpu.VMEM((B_PER_W,), jnp.float32),
            pltpu.VMEM((B_PER_W,), jnp.float32),
            pltpu.SemaphoreType.DMA,
        ],
    )
    return run(uidx, pidx, nidx, user_table, item_table)
